# baseline (speedup 1.0000x reference)
import functools

import jax
import jax.numpy as jnp
from jax.experimental import pallas as pl
from jax.experimental.pallas import tpu as pltpu

BN_EPS = 1e-5


def _conv_stage_kernel(xb_ref, eye_ref, scale_ref, shift_ref, w_ref, b_ref,
                       y_ref, s_ref, ss_ref, scr_ref,
                       *, act_input, nchw_in, hw, tr):
    H, W = hw
    Co = w_ref.shape[-1] // 3
    Ci = w_ref.shape[0] // 3

    if nchw_in:
        rows = 8
        for h0 in range(0, H, rows):
            chunk = jax.lax.dot_general(
                xb_ref[0][:, h0 * W:(h0 + rows) * W].astype(jnp.bfloat16),
                eye_ref[...],
                dimension_numbers=(((0,), (0,)), ((), ())),
                preferred_element_type=jnp.float32)
            scr_ref[h0:h0 + rows, :, Ci:2 * Ci] = (
                chunk.astype(jnp.bfloat16).reshape(rows, W, Ci))
        scr_ref[1:H, :, 0:Ci] = scr_ref[0:H - 1, :, Ci:2 * Ci]
        scr_ref[0:1, :, 0:Ci] = jnp.zeros((1, W, Ci), jnp.bfloat16)
        scr_ref[0:H - 1, :, 2 * Ci:3 * Ci] = scr_ref[1:H, :, Ci:2 * Ci]
        scr_ref[H - 1:H, :, 2 * Ci:3 * Ci] = jnp.zeros((1, W, Ci), jnp.bfloat16)
    else:
        xb = xb_ref[0]
        if act_input:
            sc = scale_ref[...].reshape(1, 1, Ci)
            sh = shift_ref[...].reshape(1, 1, Ci)
            xb = jnp.maximum(xb.astype(jnp.float32) * sc + sh, 0.0)
        xb = xb.astype(jnp.bfloat16)
        scr_ref[:, :, Ci:2 * Ci] = xb
        scr_ref[1:H, :, 0:Ci] = xb[0:H - 1]
        scr_ref[0:1, :, 0:Ci] = jnp.zeros((1, W, Ci), jnp.bfloat16)
        scr_ref[0:H - 1, :, 2 * Ci:3 * Ci] = xb[1:H]
        scr_ref[H - 1:H, :, 2 * Ci:3 * Ci] = jnp.zeros((1, W, Ci), jnp.bfloat16)

    bias = b_ref[...]
    M = tr * W
    iota = jax.lax.broadcasted_iota(jnp.int32, (M, 1), 0)
    mask_l = (iota % W != 0).astype(jnp.float32)
    mask_r = (iota % W != W - 1).astype(jnp.float32)
    s_tot = jnp.zeros((1, Co), jnp.float32)
    ss_tot = jnp.zeros((1, Co), jnp.float32)
    for r0 in range(0, H, tr):
        lhs = scr_ref[r0:r0 + tr].reshape(M, 3 * Ci)
        pair = jnp.dot(lhs, w_ref[:, 0:2 * Co],
                       preferred_element_type=jnp.float32)
        acc = jnp.dot(lhs, w_ref[:, 2 * Co:3 * Co],
                      preferred_element_type=jnp.float32)
        acc = acc + jnp.roll(pair[:, 0:Co], 1, axis=0) * mask_l
        acc = acc + jnp.roll(pair[:, Co:2 * Co], -1, axis=0) * mask_r
        acc = acc + bias
        y_ref[0, r0:r0 + tr, :, :] = acc.reshape(tr, W, Co).astype(jnp.bfloat16)
        s_tot = s_tot + jnp.sum(acc, axis=0, keepdims=True)
        ss_tot = ss_tot + jnp.sum(acc * acc, axis=0, keepdims=True)

    s_ref[...] = jnp.broadcast_to(s_tot.reshape(1, 1, Co), (1, 8, Co))
    ss_ref[...] = jnp.broadcast_to(ss_tot.reshape(1, 1, Co), (1, 8, Co))


def _conv_stage(x, eye, scale, shift, w_packed, b, *, act_input, nchw_in, hw,
                tr):
    N = x.shape[0]
    H, W = hw
    Ci = w_packed.shape[0] // 3
    Co = w_packed.shape[-1] // 3

    x_spec = (pl.BlockSpec((1, Ci, H * W), lambda n: (n, 0, 0)) if nchw_in
              else pl.BlockSpec((1, H, W, Ci), lambda n: (n, 0, 0, 0)))
    body = functools.partial(_conv_stage_kernel, act_input=act_input,
                             nchw_in=nchw_in, hw=hw, tr=tr)
    return pl.pallas_call(
        body,
        grid=(N,),
        in_specs=[
            x_spec,
            pl.BlockSpec((Ci, Ci), lambda n: (0, 0)),
            pl.BlockSpec((1, Ci), lambda n: (0, 0)),
            pl.BlockSpec((1, Ci), lambda n: (0, 0)),
            pl.BlockSpec((3 * Ci, 3 * Co), lambda n: (0, 0)),
            pl.BlockSpec((1, Co), lambda n: (0, 0)),
        ],
        out_specs=(
            pl.BlockSpec((1, H, W, Co), lambda n: (n, 0, 0, 0)),
            pl.BlockSpec((1, 8, Co), lambda n: (n, 0, 0)),
            pl.BlockSpec((1, 8, Co), lambda n: (n, 0, 0)),
        ),
        out_shape=(
            jax.ShapeDtypeStruct((N, H, W, Co), jnp.bfloat16),
            jax.ShapeDtypeStruct((N, 8, Co), jnp.float32),
            jax.ShapeDtypeStruct((N, 8, Co), jnp.float32),
        ),
        scratch_shapes=[
            pltpu.VMEM((H, W, 3 * Ci), jnp.bfloat16),
        ],
        compiler_params=pltpu.CompilerParams(
            dimension_semantics=("parallel",),
            vmem_limit_bytes=48 * 1024 * 1024),
    )(x, eye, scale, shift, w_packed, b)


def _norm_relu_kernel(y_ref, scale_ref, shift_ref, o_ref):
    C = y_ref.shape[-1]
    sc = scale_ref[...].reshape(1, 1, 1, C)
    sh = shift_ref[...].reshape(1, 1, 1, C)
    v = jnp.maximum(y_ref[...].astype(jnp.float32) * sc + sh, 0.0)
    o_ref[...] = v.astype(jnp.bfloat16)


def _norm_relu(y, scale, shift):
    N, H, W, C = y.shape
    return pl.pallas_call(
        _norm_relu_kernel,
        grid=(N,),
        in_specs=[
            pl.BlockSpec((1, H, W, C), lambda n: (n, 0, 0, 0)),
            pl.BlockSpec((1, C), lambda n: (0, 0)),
            pl.BlockSpec((1, C), lambda n: (0, 0)),
        ],
        out_specs=pl.BlockSpec((1, H, W, C), lambda n: (n, 0, 0, 0)),
        out_shape=jax.ShapeDtypeStruct((N, H, W, C), jnp.bfloat16),
        compiler_params=pltpu.CompilerParams(
            dimension_semantics=("parallel",),
            vmem_limit_bytes=32 * 1024 * 1024),
    )(y, scale, shift)


def _bn_affine(s_part, ss_part, gamma, beta, cnt, total):
    C = s_part.shape[-1]
    s = s_part.reshape(-1, C)
    ss = ss_part.reshape(-1, C)
    mean_p = s / cnt
    m2_p = ss - s * mean_p
    mean = jnp.sum(s, axis=0) / total
    m2 = jnp.sum(m2_p, axis=0) + cnt * jnp.sum((mean_p - mean) ** 2, axis=0)
    var = m2 / total
    scale = gamma.reshape(-1) * jax.lax.rsqrt(var + BN_EPS)
    shift = beta.reshape(-1) - mean * scale
    return scale.reshape(1, C), shift.reshape(1, C)


def _pack_w(w):
    slabs = [jnp.concatenate([w[dy, dx] for dy in range(3)], axis=0)
             for dx in range(3)]
    return jnp.concatenate([slabs[0], slabs[2], slabs[1]],
                           axis=1).astype(jnp.bfloat16)


def kernel(x, w1, b1, g1, be1, w2, b2, g2, be2):
    N, Cin, H, W = x.shape
    Cout = w1.shape[-1]
    tr = 4 if (H % 4 == 0) else 1

    x_flat = x.reshape(N, Cin, H * W)
    eye = jnp.eye(Cin, dtype=jnp.bfloat16)
    w1p = _pack_w(w1)
    w2p = _pack_w(w2)
    b1r = b1.reshape(1, Cout).astype(jnp.float32)
    b2r = b2.reshape(1, Cout).astype(jnp.float32)
    no_aff = jnp.zeros((1, Cin), jnp.float32)

    cnt = float(H * W)
    total = float(N * H * W)

    y1, s1, ss1 = _conv_stage(x_flat, eye, no_aff, no_aff, w1p, b1r,
                              act_input=False, nchw_in=True, hw=(H, W), tr=tr)
    sc1, sh1 = _bn_affine(s1[:, 0, :], ss1[:, 0, :], g1, be1, cnt, total)

    y2, s2, ss2 = _conv_stage(y1, eye, sc1, sh1, w2p, b2r,
                              act_input=True, nchw_in=False, hw=(H, W), tr=tr)
    sc2, sh2 = _bn_affine(s2[:, 0, :], ss2[:, 0, :], g2, be2, cnt, total)

    out = _norm_relu(y2, sc2, sh2)
    return jnp.transpose(out, (0, 3, 1, 2)).astype(jnp.float32)


# speedup vs baseline: 1.2237x; 1.2237x over previous
import functools

import jax
import jax.numpy as jnp
from jax.experimental import pallas as pl
from jax.experimental.pallas import tpu as pltpu

BN_EPS = 1e-5


def _conv_stage_kernel(xb_ref, scale_ref, shift_ref, w_ref, b_ref,
                       y_ref, s_ref, ss_ref, scr_ref, *, act_input, tr):
    _, H, W, Ci = xb_ref.shape
    Co = w_ref.shape[-1] // 3

    xb = xb_ref[0]
    if act_input:
        sc = scale_ref[...].reshape(1, 1, Ci)
        sh = shift_ref[...].reshape(1, 1, Ci)
        xb = jnp.maximum(xb.astype(jnp.float32) * sc + sh, 0.0)
    xb = xb.astype(jnp.bfloat16)
    scr_ref[:, :, Ci:2 * Ci] = xb
    scr_ref[1:H, :, 0:Ci] = xb[0:H - 1]
    scr_ref[0:1, :, 0:Ci] = jnp.zeros((1, W, Ci), jnp.bfloat16)
    scr_ref[0:H - 1, :, 2 * Ci:3 * Ci] = xb[1:H]
    scr_ref[H - 1:H, :, 2 * Ci:3 * Ci] = jnp.zeros((1, W, Ci), jnp.bfloat16)

    bias = b_ref[...]
    M = tr * W
    iota = jax.lax.broadcasted_iota(jnp.int32, (M, 1), 0)
    mask_l = (iota % W != 0).astype(jnp.float32)
    mask_r = (iota % W != W - 1).astype(jnp.float32)
    s_tot = jnp.zeros((1, Co), jnp.float32)
    ss_tot = jnp.zeros((1, Co), jnp.float32)
    for r0 in range(0, H, tr):
        lhs = scr_ref[r0:r0 + tr].reshape(M, 3 * Ci)
        pair = jnp.dot(lhs, w_ref[:, 0:2 * Co],
                       preferred_element_type=jnp.float32)
        acc = jnp.dot(lhs, w_ref[:, 2 * Co:3 * Co],
                      preferred_element_type=jnp.float32)
        acc = acc + jnp.roll(pair[:, 0:Co], 1, axis=0) * mask_l
        acc = acc + jnp.roll(pair[:, Co:2 * Co], -1, axis=0) * mask_r
        acc = acc + bias
        y_ref[0, r0:r0 + tr, :, :] = acc.reshape(tr, W, Co).astype(jnp.bfloat16)
        s_tot = s_tot + jnp.sum(acc, axis=0, keepdims=True)
        ss_tot = ss_tot + jnp.sum(acc * acc, axis=0, keepdims=True)

    s_ref[...] = jnp.broadcast_to(s_tot.reshape(1, 1, Co), (1, 8, Co))
    ss_ref[...] = jnp.broadcast_to(ss_tot.reshape(1, 1, Co), (1, 8, Co))


def _conv_stage(x, scale, shift, w_packed, b, *, act_input, tr):
    N, H, W, Ci = x.shape
    Co = w_packed.shape[-1] // 3

    body = functools.partial(_conv_stage_kernel, act_input=act_input, tr=tr)
    return pl.pallas_call(
        body,
        grid=(N,),
        in_specs=[
            pl.BlockSpec((1, H, W, Ci), lambda n: (n, 0, 0, 0)),
            pl.BlockSpec((1, Ci), lambda n: (0, 0)),
            pl.BlockSpec((1, Ci), lambda n: (0, 0)),
            pl.BlockSpec((3 * Ci, 3 * Co), lambda n: (0, 0)),
            pl.BlockSpec((1, Co), lambda n: (0, 0)),
        ],
        out_specs=(
            pl.BlockSpec((1, H, W, Co), lambda n: (n, 0, 0, 0)),
            pl.BlockSpec((1, 8, Co), lambda n: (n, 0, 0)),
            pl.BlockSpec((1, 8, Co), lambda n: (n, 0, 0)),
        ),
        out_shape=(
            jax.ShapeDtypeStruct((N, H, W, Co), jnp.bfloat16),
            jax.ShapeDtypeStruct((N, 8, Co), jnp.float32),
            jax.ShapeDtypeStruct((N, 8, Co), jnp.float32),
        ),
        scratch_shapes=[
            pltpu.VMEM((H, W, 3 * Ci), jnp.bfloat16),
        ],
        compiler_params=pltpu.CompilerParams(
            dimension_semantics=("parallel",),
            vmem_limit_bytes=48 * 1024 * 1024),
    )(x, scale, shift, w_packed, b)


def _norm_relu_kernel(y_ref, scale_ref, shift_ref, o_ref):
    C = y_ref.shape[-1]
    sc = scale_ref[...].reshape(1, 1, 1, C)
    sh = shift_ref[...].reshape(1, 1, 1, C)
    v = jnp.maximum(y_ref[...].astype(jnp.float32) * sc + sh, 0.0)
    o_ref[...] = v.astype(jnp.bfloat16)


def _norm_relu(y, scale, shift):
    N, H, W, C = y.shape
    nb = 2 if N % 2 == 0 else 1
    return pl.pallas_call(
        _norm_relu_kernel,
        grid=(N // nb,),
        in_specs=[
            pl.BlockSpec((nb, H, W, C), lambda n: (n, 0, 0, 0)),
            pl.BlockSpec((1, C), lambda n: (0, 0)),
            pl.BlockSpec((1, C), lambda n: (0, 0)),
        ],
        out_specs=pl.BlockSpec((nb, H, W, C), lambda n: (n, 0, 0, 0)),
        out_shape=jax.ShapeDtypeStruct((N, H, W, C), jnp.bfloat16),
        compiler_params=pltpu.CompilerParams(
            dimension_semantics=("parallel",),
            vmem_limit_bytes=32 * 1024 * 1024),
    )(y, scale, shift)


def _bn_affine(s_part, ss_part, gamma, beta, cnt, total):
    C = s_part.shape[-1]
    s = s_part.reshape(-1, C)
    ss = ss_part.reshape(-1, C)
    mean_p = s / cnt
    m2_p = ss - s * mean_p
    mean = jnp.sum(s, axis=0) / total
    m2 = jnp.sum(m2_p, axis=0) + cnt * jnp.sum((mean_p - mean) ** 2, axis=0)
    var = m2 / total
    scale = gamma.reshape(-1) * jax.lax.rsqrt(var + BN_EPS)
    shift = beta.reshape(-1) - mean * scale
    return scale.reshape(1, C), shift.reshape(1, C)


def _pack_w(w):
    slabs = [jnp.concatenate([w[dy, dx] for dy in range(3)], axis=0)
             for dx in range(3)]
    return jnp.concatenate([slabs[0], slabs[2], slabs[1]],
                           axis=1).astype(jnp.bfloat16)


def kernel(x, w1, b1, g1, be1, w2, b2, g2, be2):
    N, Cin, H, W = x.shape
    Cout = w1.shape[-1]
    tr = 4 if (H % 4 == 0) else 1

    xh = jnp.transpose(x, (0, 2, 3, 1)).astype(jnp.bfloat16)

    w1p = _pack_w(w1)
    w2p = _pack_w(w2)
    b1r = b1.reshape(1, Cout).astype(jnp.float32)
    b2r = b2.reshape(1, Cout).astype(jnp.float32)
    no_aff = jnp.zeros((1, Cin), jnp.float32)

    cnt = float(H * W)
    total = float(N * H * W)

    y1, s1, ss1 = _conv_stage(xh, no_aff, no_aff, w1p, b1r,
                              act_input=False, tr=tr)
    sc1, sh1 = _bn_affine(s1[:, 0, :], ss1[:, 0, :], g1, be1, cnt, total)

    y2, s2, ss2 = _conv_stage(y1, sc1, sh1, w2p, b2r,
                              act_input=True, tr=tr)
    sc2, sh2 = _bn_affine(s2[:, 0, :], ss2[:, 0, :], g2, be2, cnt, total)

    out = _norm_relu(y2, sc2, sh2)
    return jnp.transpose(out, (0, 3, 1, 2)).astype(jnp.float32)


# speedup vs baseline: 1.2334x; 1.0079x over previous
import functools

import jax
import jax.numpy as jnp
from jax.experimental import pallas as pl
from jax.experimental.pallas import tpu as pltpu

BN_EPS = 1e-5


def _conv_stage_kernel(xb_ref, scale_ref, shift_ref, w_ref, b_ref,
                       y_ref, s_ref, ss_ref, scr_ref, *, act_input, tr):
    nb, H, W, Ci = xb_ref.shape
    Co = w_ref.shape[-1] // 3
    M = tr * W
    iota = jax.lax.broadcasted_iota(jnp.int32, (M, 1), 0)
    mask_l = (iota % W != 0).astype(jnp.float32)
    mask_r = (iota % W != W - 1).astype(jnp.float32)
    bias = b_ref[...]

    for img in range(nb):
        xb = xb_ref[img]
        if act_input:
            sc = scale_ref[...].reshape(1, 1, Ci)
            sh = shift_ref[...].reshape(1, 1, Ci)
            xb = jnp.maximum(xb.astype(jnp.float32) * sc + sh, 0.0)
        xb = xb.astype(jnp.bfloat16)
        scr_ref[:, :, Ci:2 * Ci] = xb
        scr_ref[1:H, :, 0:Ci] = xb[0:H - 1]
        scr_ref[0:1, :, 0:Ci] = jnp.zeros((1, W, Ci), jnp.bfloat16)
        scr_ref[0:H - 1, :, 2 * Ci:3 * Ci] = xb[1:H]
        scr_ref[H - 1:H, :, 2 * Ci:3 * Ci] = jnp.zeros((1, W, Ci), jnp.bfloat16)

        s_tot = jnp.zeros((1, Co), jnp.float32)
        ss_tot = jnp.zeros((1, Co), jnp.float32)
        for r0 in range(0, H, tr):
            lhs = scr_ref[r0:r0 + tr].reshape(M, 3 * Ci)
            pair = jnp.dot(lhs, w_ref[:, 0:2 * Co],
                           preferred_element_type=jnp.float32)
            acc = jnp.dot(lhs, w_ref[:, 2 * Co:3 * Co],
                          preferred_element_type=jnp.float32)
            acc = acc + jnp.roll(pair[:, 0:Co], 1, axis=0) * mask_l
            acc = acc + jnp.roll(pair[:, Co:2 * Co], -1, axis=0) * mask_r
            acc = acc + bias
            y_ref[img, r0:r0 + tr, :, :] = (
                acc.reshape(tr, W, Co).astype(jnp.bfloat16))
            s_tot = s_tot + jnp.sum(acc, axis=0, keepdims=True)
            ss_tot = ss_tot + jnp.sum(acc * acc, axis=0, keepdims=True)

        s_ref[img] = jnp.broadcast_to(s_tot.reshape(1, Co), (8, Co))
        ss_ref[img] = jnp.broadcast_to(ss_tot.reshape(1, Co), (8, Co))


def _conv_stage(x, scale, shift, w_packed, b, *, act_input, tr):
    N, H, W, Ci = x.shape
    Co = w_packed.shape[-1] // 3

    nb = 2 if N % 2 == 0 else 1
    body = functools.partial(_conv_stage_kernel, act_input=act_input, tr=tr)
    return pl.pallas_call(
        body,
        grid=(N // nb,),
        in_specs=[
            pl.BlockSpec((nb, H, W, Ci), lambda n: (n, 0, 0, 0)),
            pl.BlockSpec((1, Ci), lambda n: (0, 0)),
            pl.BlockSpec((1, Ci), lambda n: (0, 0)),
            pl.BlockSpec((3 * Ci, 3 * Co), lambda n: (0, 0)),
            pl.BlockSpec((1, Co), lambda n: (0, 0)),
        ],
        out_specs=(
            pl.BlockSpec((nb, H, W, Co), lambda n: (n, 0, 0, 0)),
            pl.BlockSpec((nb, 8, Co), lambda n: (n, 0, 0)),
            pl.BlockSpec((nb, 8, Co), lambda n: (n, 0, 0)),
        ),
        out_shape=(
            jax.ShapeDtypeStruct((N, H, W, Co), jnp.bfloat16),
            jax.ShapeDtypeStruct((N, 8, Co), jnp.float32),
            jax.ShapeDtypeStruct((N, 8, Co), jnp.float32),
        ),
        scratch_shapes=[
            pltpu.VMEM((H, W, 3 * Ci), jnp.bfloat16),
        ],
        compiler_params=pltpu.CompilerParams(
            dimension_semantics=("parallel",),
            vmem_limit_bytes=48 * 1024 * 1024),
    )(x, scale, shift, w_packed, b)


def _norm_relu_kernel(y_ref, scale_ref, shift_ref, o_ref):
    C = y_ref.shape[-1]
    sc = scale_ref[...].reshape(1, 1, 1, C)
    sh = shift_ref[...].reshape(1, 1, 1, C)
    v = jnp.maximum(y_ref[...].astype(jnp.float32) * sc + sh, 0.0)
    o_ref[...] = v.astype(jnp.bfloat16)


def _norm_relu(y, scale, shift):
    N, H, W, C = y.shape
    nb = 2 if N % 2 == 0 else 1
    return pl.pallas_call(
        _norm_relu_kernel,
        grid=(N // nb,),
        in_specs=[
            pl.BlockSpec((nb, H, W, C), lambda n: (n, 0, 0, 0)),
            pl.BlockSpec((1, C), lambda n: (0, 0)),
            pl.BlockSpec((1, C), lambda n: (0, 0)),
        ],
        out_specs=pl.BlockSpec((nb, H, W, C), lambda n: (n, 0, 0, 0)),
        out_shape=jax.ShapeDtypeStruct((N, H, W, C), jnp.bfloat16),
        compiler_params=pltpu.CompilerParams(
            dimension_semantics=("parallel",),
            vmem_limit_bytes=32 * 1024 * 1024),
    )(y, scale, shift)


def _bn_affine(s_part, ss_part, gamma, beta, cnt, total):
    C = s_part.shape[-1]
    s = s_part.reshape(-1, C)
    ss = ss_part.reshape(-1, C)
    mean_p = s / cnt
    m2_p = ss - s * mean_p
    mean = jnp.sum(s, axis=0) / total
    m2 = jnp.sum(m2_p, axis=0) + cnt * jnp.sum((mean_p - mean) ** 2, axis=0)
    var = m2 / total
    scale = gamma.reshape(-1) * jax.lax.rsqrt(var + BN_EPS)
    shift = beta.reshape(-1) - mean * scale
    return scale.reshape(1, C), shift.reshape(1, C)


def _pack_w(w):
    slabs = [jnp.concatenate([w[dy, dx] for dy in range(3)], axis=0)
             for dx in range(3)]
    return jnp.concatenate([slabs[0], slabs[2], slabs[1]],
                           axis=1).astype(jnp.bfloat16)


def kernel(x, w1, b1, g1, be1, w2, b2, g2, be2):
    N, Cin, H, W = x.shape
    Cout = w1.shape[-1]
    tr = 4 if (H % 4 == 0) else 1

    xh = jnp.transpose(x, (0, 2, 3, 1)).astype(jnp.bfloat16)

    w1p = _pack_w(w1)
    w2p = _pack_w(w2)
    b1r = b1.reshape(1, Cout).astype(jnp.float32)
    b2r = b2.reshape(1, Cout).astype(jnp.float32)
    no_aff = jnp.zeros((1, Cin), jnp.float32)

    cnt = float(H * W)
    total = float(N * H * W)

    y1, s1, ss1 = _conv_stage(xh, no_aff, no_aff, w1p, b1r,
                              act_input=False, tr=tr)
    sc1, sh1 = _bn_affine(s1[:, 0, :], ss1[:, 0, :], g1, be1, cnt, total)

    y2, s2, ss2 = _conv_stage(y1, sc1, sh1, w2p, b2r,
                              act_input=True, tr=tr)
    sc2, sh2 = _bn_affine(s2[:, 0, :], ss2[:, 0, :], g2, be2, cnt, total)

    out = _norm_relu(y2, sc2, sh2)
    return jnp.transpose(out, (0, 3, 1, 2)).astype(jnp.float32)


# speedup vs baseline: 1.2365x; 1.0025x over previous
import functools

import jax
import jax.numpy as jnp
from jax.experimental import pallas as pl
from jax.experimental.pallas import tpu as pltpu

BN_EPS = 1e-5


def _conv_stage_kernel(xb_ref, scale_ref, shift_ref, w_ref, b_ref,
                       y_ref, s_ref, ss_ref, scr_ref, *, act_input, tr):
    nb, H, W, Ci = xb_ref.shape
    Co = w_ref.shape[-1] // 3
    M = tr * W
    iota = jax.lax.broadcasted_iota(jnp.int32, (M, 1), 0)
    mask_l = (iota % W != 0).astype(jnp.float32)
    mask_r = (iota % W != W - 1).astype(jnp.float32)
    bias = b_ref[...]

    for img in range(nb):
        xb = xb_ref[img]
        if act_input:
            sc = scale_ref[...].reshape(1, 1, Ci)
            sh = shift_ref[...].reshape(1, 1, Ci)
            xb = jnp.maximum(xb.astype(jnp.float32) * sc + sh, 0.0)
        xb = xb.astype(jnp.bfloat16)
        scr_ref[:, :, Ci:2 * Ci] = xb
        scr_ref[1:H, :, 0:Ci] = xb[0:H - 1]
        scr_ref[0:1, :, 0:Ci] = jnp.zeros((1, W, Ci), jnp.bfloat16)
        scr_ref[0:H - 1, :, 2 * Ci:3 * Ci] = xb[1:H]
        scr_ref[H - 1:H, :, 2 * Ci:3 * Ci] = jnp.zeros((1, W, Ci), jnp.bfloat16)

        s_tot = jnp.zeros((1, Co), jnp.float32)
        ss_tot = jnp.zeros((1, Co), jnp.float32)
        for r0 in range(0, H, tr):
            lhs = scr_ref[r0:r0 + tr].reshape(M, 3 * Ci)
            pair = jnp.dot(lhs, w_ref[:, 0:2 * Co],
                           preferred_element_type=jnp.float32)
            acc = jnp.dot(lhs, w_ref[:, 2 * Co:3 * Co],
                          preferred_element_type=jnp.float32)
            acc = acc + jnp.roll(pair[:, 0:Co], 1, axis=0) * mask_l
            acc = acc + jnp.roll(pair[:, Co:2 * Co], -1, axis=0) * mask_r
            acc = acc + bias
            y_ref[img, r0:r0 + tr, :, :] = (
                acc.reshape(tr, W, Co).astype(jnp.bfloat16))
            s_tot = s_tot + jnp.sum(acc, axis=0, keepdims=True)
            ss_tot = ss_tot + jnp.sum(acc * acc, axis=0, keepdims=True)

        s_ref[img] = jnp.broadcast_to(s_tot.reshape(1, Co), (8, Co))
        ss_ref[img] = jnp.broadcast_to(ss_tot.reshape(1, Co), (8, Co))


def _conv_stage(x, scale, shift, w_packed, b, *, act_input, tr):
    N, H, W, Ci = x.shape
    Co = w_packed.shape[-1] // 3

    nb = 4 if N % 4 == 0 else 1
    body = functools.partial(_conv_stage_kernel, act_input=act_input, tr=tr)
    return pl.pallas_call(
        body,
        grid=(N // nb,),
        in_specs=[
            pl.BlockSpec((nb, H, W, Ci), lambda n: (n, 0, 0, 0)),
            pl.BlockSpec((1, Ci), lambda n: (0, 0)),
            pl.BlockSpec((1, Ci), lambda n: (0, 0)),
            pl.BlockSpec((3 * Ci, 3 * Co), lambda n: (0, 0)),
            pl.BlockSpec((1, Co), lambda n: (0, 0)),
        ],
        out_specs=(
            pl.BlockSpec((nb, H, W, Co), lambda n: (n, 0, 0, 0)),
            pl.BlockSpec((nb, 8, Co), lambda n: (n, 0, 0)),
            pl.BlockSpec((nb, 8, Co), lambda n: (n, 0, 0)),
        ),
        out_shape=(
            jax.ShapeDtypeStruct((N, H, W, Co), jnp.bfloat16),
            jax.ShapeDtypeStruct((N, 8, Co), jnp.float32),
            jax.ShapeDtypeStruct((N, 8, Co), jnp.float32),
        ),
        scratch_shapes=[
            pltpu.VMEM((H, W, 3 * Ci), jnp.bfloat16),
        ],
        compiler_params=pltpu.CompilerParams(
            dimension_semantics=("parallel",),
            vmem_limit_bytes=48 * 1024 * 1024),
    )(x, scale, shift, w_packed, b)


def _norm_relu_kernel(y_ref, scale_ref, shift_ref, o_ref):
    C = y_ref.shape[-1]
    sc = scale_ref[...].reshape(1, 1, 1, C)
    sh = shift_ref[...].reshape(1, 1, 1, C)
    v = jnp.maximum(y_ref[...].astype(jnp.float32) * sc + sh, 0.0)
    o_ref[...] = v.astype(jnp.bfloat16)


def _norm_relu(y, scale, shift):
    N, H, W, C = y.shape
    nb = 4 if N % 4 == 0 else 1
    return pl.pallas_call(
        _norm_relu_kernel,
        grid=(N // nb,),
        in_specs=[
            pl.BlockSpec((nb, H, W, C), lambda n: (n, 0, 0, 0)),
            pl.BlockSpec((1, C), lambda n: (0, 0)),
            pl.BlockSpec((1, C), lambda n: (0, 0)),
        ],
        out_specs=pl.BlockSpec((nb, H, W, C), lambda n: (n, 0, 0, 0)),
        out_shape=jax.ShapeDtypeStruct((N, H, W, C), jnp.bfloat16),
        compiler_params=pltpu.CompilerParams(
            dimension_semantics=("parallel",),
            vmem_limit_bytes=32 * 1024 * 1024),
    )(y, scale, shift)


def _bn_affine(s_part, ss_part, gamma, beta, cnt, total):
    C = s_part.shape[-1]
    s = s_part.reshape(-1, C)
    ss = ss_part.reshape(-1, C)
    mean_p = s / cnt
    m2_p = ss - s * mean_p
    mean = jnp.sum(s, axis=0) / total
    m2 = jnp.sum(m2_p, axis=0) + cnt * jnp.sum((mean_p - mean) ** 2, axis=0)
    var = m2 / total
    scale = gamma.reshape(-1) * jax.lax.rsqrt(var + BN_EPS)
    shift = beta.reshape(-1) - mean * scale
    return scale.reshape(1, C), shift.reshape(1, C)


def _pack_w(w):
    slabs = [jnp.concatenate([w[dy, dx] for dy in range(3)], axis=0)
             for dx in range(3)]
    return jnp.concatenate([slabs[0], slabs[2], slabs[1]],
                           axis=1).astype(jnp.bfloat16)


def kernel(x, w1, b1, g1, be1, w2, b2, g2, be2):
    N, Cin, H, W = x.shape
    Cout = w1.shape[-1]
    tr = 4 if (H % 4 == 0) else 1

    xh = jnp.transpose(x, (0, 2, 3, 1)).astype(jnp.bfloat16)

    w1p = _pack_w(w1)
    w2p = _pack_w(w2)
    b1r = b1.reshape(1, Cout).astype(jnp.float32)
    b2r = b2.reshape(1, Cout).astype(jnp.float32)
    no_aff = jnp.zeros((1, Cin), jnp.float32)

    cnt = float(H * W)
    total = float(N * H * W)

    y1, s1, ss1 = _conv_stage(xh, no_aff, no_aff, w1p, b1r,
                              act_input=False, tr=tr)
    sc1, sh1 = _bn_affine(s1[:, 0, :], ss1[:, 0, :], g1, be1, cnt, total)

    y2, s2, ss2 = _conv_stage(y1, sc1, sh1, w2p, b2r,
                              act_input=True, tr=tr)
    sc2, sh2 = _bn_affine(s2[:, 0, :], ss2[:, 0, :], g2, be2, cnt, total)

    out = _norm_relu(y2, sc2, sh2)
    return jnp.transpose(out, (0, 3, 1, 2)).astype(jnp.float32)
